# baseline (speedup 1.0000x reference)
import functools

import jax
import jax.numpy as jnp
from jax import lax
from jax.experimental import pallas as pl
from jax.experimental.pallas import tpu as pltpu
from jax.experimental.pallas import tpu_sc as plsc

DIM = 1024
SCALE = DIM ** (-0.5)

_NC = 2
_NS = 16
_NW = _NC * _NS

_TOTAL = 8192 * DIM
_PER_W = _TOTAL // _NW
_CHUNK = 16 * DIM
_NBUF = 4
_NCHUNK = _PER_W // _CHUNK
_LAG = 2


def _sc_body(w_hbm, out_hbm, buf, in_sem, out_sem):
    wid = lax.axis_index("s") * _NC + lax.axis_index("c")
    base = wid * _PER_W

    def start_in(i):
        return pltpu.async_copy(
            w_hbm.at[pl.ds(base + i * _CHUNK, _CHUNK)],
            buf.at[i % _NBUF],
            in_sem.at[i % _NBUF],
        )

    def start_out(i):
        return pltpu.async_copy(
            buf.at[i % _NBUF],
            out_hbm.at[pl.ds(base + i * _CHUNK, _CHUNK)],
            out_sem.at[i % _NBUF],
        )

    in_descs = {}
    out_descs = {}
    unwaited_out = set()

    for i in range(_NBUF):
        in_descs[i] = start_in(i)

    for i in range(_NCHUNK):
        b = i % _NBUF
        j = i - _LAG
        if j >= 0 and j + _NBUF < _NCHUNK:
            out_descs[j].wait()
            unwaited_out.discard(j)
            in_descs[j + _NBUF] = start_in(j + _NBUF)

        in_descs[i].wait()

        @plsc.parallel_loop(0, _CHUNK, 16, unroll=8)
        def _(k):
            buf[b, pl.ds(k, 16)] = buf[b, pl.ds(k, 16)] * SCALE

        out_descs[i] = start_out(i)
        unwaited_out.add(i)

    for i in sorted(unwaited_out):
        out_descs[i].wait()


def kernel(x, W):
    n = x.shape[1]
    w_flat = jnp.reshape(W[:n], (-1,))
    mesh = plsc.VectorSubcoreMesh(core_axis_name="c", subcore_axis_name="s")
    out = pl.kernel(
        _sc_body,
        out_type=jax.ShapeDtypeStruct((n * DIM,), jnp.float32),
        mesh=mesh,
        scratch_types=[
            pltpu.VMEM((_NBUF, _CHUNK), jnp.float32),
            pltpu.SemaphoreType.DMA((_NBUF,)),
            pltpu.SemaphoreType.DMA((_NBUF,)),
        ],
    )(w_flat)
    return jnp.reshape(out, (n, DIM))


# speedup vs baseline: 2.1917x; 2.1917x over previous
import jax
import jax.numpy as jnp
from jax import lax
from jax.experimental import pallas as pl
from jax.experimental.pallas import tpu as pltpu
from jax.experimental.pallas import tpu_sc as plsc

DIM = 1024
SCALE = DIM ** (-0.5)

_NC = 2
_NS = 16
_NW = _NC * _NS

_ROWS = 8192
_PER_W = _ROWS // _NW
_CHUNK = 16
_NBUF = 4
_NCHUNK = _PER_W // _CHUNK
_LAG = 2


def _sc_body(w_hbm, out_hbm, buf, in_sem, out_sem):
    wid = lax.axis_index("s") * _NC + lax.axis_index("c")
    row0 = wid * _PER_W

    def start_in(i):
        b = i % _NBUF
        return pltpu.async_copy(
            w_hbm.at[pl.ds(row0 + i * _CHUNK, _CHUNK)],
            buf.at[b],
            in_sem.at[b],
        )

    def start_out(i):
        b = i % _NBUF
        return pltpu.async_copy(
            buf.at[b],
            out_hbm.at[pl.ds(row0 + i * _CHUNK, _CHUNK)],
            out_sem.at[b],
        )

    in_descs = {}
    out_descs = {}
    unwaited_out = set()

    for i in range(_NBUF):
        in_descs[i] = start_in(i)

    for i in range(_NCHUNK):
        b = i % _NBUF
        j = i - _LAG
        if j >= 0 and j + _NBUF < _NCHUNK:
            out_descs[j].wait()
            unwaited_out.discard(j)
            in_descs[j + _NBUF] = start_in(j + _NBUF)

        in_descs[i].wait()

        @plsc.parallel_loop(0, _CHUNK, 1)
        def _(r):
            for c in range(0, DIM, 16):
                buf[b, r, pl.ds(c, 16)] = buf[b, r, pl.ds(c, 16)] * SCALE

        out_descs[i] = start_out(i)
        unwaited_out.add(i)

    for i in sorted(unwaited_out):
        out_descs[i].wait()


def kernel(x, W):
    n = x.shape[1]
    mesh = plsc.VectorSubcoreMesh(core_axis_name="c", subcore_axis_name="s")
    return pl.kernel(
        _sc_body,
        out_type=jax.ShapeDtypeStruct((n, DIM), jnp.float32),
        mesh=mesh,
        scratch_types=[
            pltpu.VMEM((_NBUF, _CHUNK, DIM), jnp.float32),
            pltpu.SemaphoreType.DMA((_NBUF,)),
            pltpu.SemaphoreType.DMA((_NBUF,)),
        ],
    )(W[:n])


# speedup vs baseline: 2.4305x; 1.1090x over previous
import jax
import jax.numpy as jnp
from jax import lax
from jax.experimental import pallas as pl
from jax.experimental.pallas import tpu as pltpu
from jax.experimental.pallas import tpu_sc as plsc

DIM = 1024
SCALE = DIM ** (-0.5)

_NC = 2
_NS = 16
_NW = _NC * _NS

_ROWS = 8192
_PER_W = _ROWS // _NW
_CHUNK = 32
_NBUF = 3
_NCHUNK = _PER_W // _CHUNK
_LAG = 1


def _sc_body(w_hbm, out_hbm, buf, in_sem, out_sem):
    wid = lax.axis_index("s") * _NC + lax.axis_index("c")
    row0 = wid * _PER_W

    def start_in(i):
        b = i % _NBUF
        return pltpu.async_copy(
            w_hbm.at[pl.ds(row0 + i * _CHUNK, _CHUNK)],
            buf.at[b],
            in_sem.at[b],
        )

    def start_out(i):
        b = i % _NBUF
        return pltpu.async_copy(
            buf.at[b],
            out_hbm.at[pl.ds(row0 + i * _CHUNK, _CHUNK)],
            out_sem.at[b],
        )

    in_descs = {}
    out_descs = {}
    unwaited_out = set()

    for i in range(_NBUF):
        in_descs[i] = start_in(i)

    for i in range(_NCHUNK):
        b = i % _NBUF
        j = i - _LAG
        if j >= 0 and j + _NBUF < _NCHUNK:
            out_descs[j].wait()
            unwaited_out.discard(j)
            in_descs[j + _NBUF] = start_in(j + _NBUF)

        in_descs[i].wait()

        @plsc.parallel_loop(0, _CHUNK, 1)
        def _(r):
            for c in range(0, DIM, 16):
                buf[b, r, pl.ds(c, 16)] = buf[b, r, pl.ds(c, 16)] * SCALE

        out_descs[i] = start_out(i)
        unwaited_out.add(i)

    for i in sorted(unwaited_out):
        out_descs[i].wait()


def kernel(x, W):
    n = x.shape[1]
    mesh = plsc.VectorSubcoreMesh(core_axis_name="c", subcore_axis_name="s")
    return pl.kernel(
        _sc_body,
        out_type=jax.ShapeDtypeStruct((n, DIM), jnp.float32),
        mesh=mesh,
        scratch_types=[
            pltpu.VMEM((_NBUF, _CHUNK, DIM), jnp.float32),
            pltpu.SemaphoreType.DMA((_NBUF,)),
            pltpu.SemaphoreType.DMA((_NBUF,)),
        ],
    )(W[:n])
